# baseline (speedup 1.0000x reference)
import functools

import jax
import jax.numpy as jnp
from jax.experimental import pallas as pl
from jax.experimental.pallas import tpu as pltpu

_ALPHA = 0.5
_NEG_LAMBDA = 1.0


def _loss_body(x_ref, y_ref, m_ref, out_ref, *, Bb, K, L, inv_count,
               num_blocks):
    x = x_ref[...]
    yb = y_ref[0]

    e16 = jnp.exp(-x).astype(jnp.bfloat16)
    e2 = e16.reshape(Bb * K, L)
    s = jax.lax.dot_general(m_ref[...], e2, (((1,), (0,)), ((), ())),
                            preferred_element_type=jnp.float32)

    xmin = jnp.min(x, axis=1)
    col = jax.lax.broadcasted_iota(jnp.int32, (Bb, L), 1)
    tmask = col == yb
    pos = jnp.min(jnp.where(tmask, xmin, jnp.inf), axis=1, keepdims=True)

    t = jnp.maximum(_ALPHA + pos + _NEG_LAMBDA * jnp.log(s), 0.0)
    partial = jnp.sum(jnp.where(tmask, 0.0, t))

    @pl.when(pl.program_id(0) == 0)
    def _init():
        out_ref[0, 0] = 0.0

    out_ref[0, 0] += partial

    @pl.when(pl.program_id(0) == num_blocks - 1)
    def _finish():
        out_ref[0, 0] = out_ref[0, 0] * inv_count


def kernel(y_hat, y):
    B, L, K = y_hat.shape
    Bb = 256
    G = B // Bb
    x_t = jnp.transpose(y_hat, (0, 2, 1))
    y3 = y.reshape(G, Bb, 1)
    m = (jnp.arange(Bb * K, dtype=jnp.int32)[None, :] // K
         == jnp.arange(Bb, dtype=jnp.int32)[:, None]).astype(jnp.bfloat16)
    total = pl.pallas_call(
        functools.partial(_loss_body, Bb=Bb, K=K, L=L,
                          inv_count=1.0 / (B * (L - 1)), num_blocks=G),
        grid=(G,),
        in_specs=[
            pl.BlockSpec((Bb, K, L), lambda i: (i, 0, 0)),
            pl.BlockSpec((1, Bb, 1), lambda i: (i, 0, 0)),
            pl.BlockSpec((Bb, Bb * K), lambda i: (0, 0)),
        ],
        out_specs=pl.BlockSpec(memory_space=pltpu.SMEM),
        out_shape=jax.ShapeDtypeStruct((1, 1), jnp.float32),
    )(x_t, y3, m)
    return total[0, 0]


# speedup vs baseline: 1.0278x; 1.0278x over previous
import functools

import jax
import jax.numpy as jnp
from jax.experimental import pallas as pl
from jax.experimental.pallas import tpu as pltpu

_ALPHA = 0.5
_NEG_LAMBDA = 1.0


def _loss_body(x_ref, y_ref, m_ref, out_ref, *, Bb, K, L, inv_count,
               num_blocks):
    x = x_ref[...]
    yb = y_ref[0]

    e16 = jnp.exp(-x).astype(jnp.bfloat16)
    e2 = e16.reshape(Bb * K, L)
    s = jax.lax.dot_general(m_ref[...], e2, (((1,), (0,)), ((), ())),
                            preferred_element_type=jnp.float32)

    col3 = jax.lax.broadcasted_iota(jnp.int32, (Bb, K, L), 2)
    xsel = jnp.where(col3 == yb.reshape(Bb, 1, 1), x, jnp.inf)
    posk = jnp.min(xsel, axis=2)
    pos = jnp.min(posk, axis=1, keepdims=True)

    col = jax.lax.broadcasted_iota(jnp.int32, (Bb, L), 1)
    tmask = col == yb

    t = jnp.maximum(_ALPHA + pos + _NEG_LAMBDA * jnp.log(s), 0.0)
    partial = jnp.sum(jnp.where(tmask, 0.0, t))

    @pl.when(pl.program_id(0) == 0)
    def _init():
        out_ref[0, 0] = 0.0

    out_ref[0, 0] += partial

    @pl.when(pl.program_id(0) == num_blocks - 1)
    def _finish():
        out_ref[0, 0] = out_ref[0, 0] * inv_count


def kernel(y_hat, y):
    B, L, K = y_hat.shape
    Bb = 256
    G = B // Bb
    x_t = jnp.transpose(y_hat, (0, 2, 1))
    y3 = y.reshape(G, Bb, 1)
    m = (jnp.arange(Bb * K, dtype=jnp.int32)[None, :] // K
         == jnp.arange(Bb, dtype=jnp.int32)[:, None]).astype(jnp.bfloat16)
    total = pl.pallas_call(
        functools.partial(_loss_body, Bb=Bb, K=K, L=L,
                          inv_count=1.0 / (B * (L - 1)), num_blocks=G),
        grid=(G,),
        in_specs=[
            pl.BlockSpec((Bb, K, L), lambda i: (i, 0, 0)),
            pl.BlockSpec((1, Bb, 1), lambda i: (i, 0, 0)),
            pl.BlockSpec((Bb, Bb * K), lambda i: (0, 0)),
        ],
        out_specs=pl.BlockSpec(memory_space=pltpu.SMEM),
        out_shape=jax.ShapeDtypeStruct((1, 1), jnp.float32),
    )(x_t, y3, m)
    return total[0, 0]


# speedup vs baseline: 1.0862x; 1.0569x over previous
import functools

import jax
import jax.numpy as jnp
from jax.experimental import pallas as pl
from jax.experimental.pallas import tpu as pltpu

_ALPHA = 0.5
_NEG_LAMBDA = 1.0


def _loss_body(x_ref, y_ref, m_ref, out_ref, *, Bb, K, L, inv_count,
               num_blocks):
    x = x_ref[...]
    yb = y_ref[0]

    e16 = jnp.exp(-x).astype(jnp.bfloat16)
    e2 = e16.reshape(Bb * K, L)
    s = jax.lax.dot_general(m_ref[...], e2, (((1,), (0,)), ((), ())),
                            preferred_element_type=jnp.float32)

    col3 = jax.lax.broadcasted_iota(jnp.int32, (Bb, K, L), 2)
    xsel = jnp.where(col3 == yb.reshape(Bb, 1, 1), x, jnp.inf)
    posk = jnp.min(xsel, axis=2)
    pos = jnp.min(posk, axis=1, keepdims=True)

    col = jax.lax.broadcasted_iota(jnp.int32, (Bb, L), 1)
    tmask = col == yb

    t = jnp.maximum(_ALPHA + pos + _NEG_LAMBDA * jnp.log(s), 0.0)
    partial = jnp.sum(jnp.where(tmask, 0.0, t))

    @pl.when(pl.program_id(0) == 0)
    def _init():
        out_ref[0, 0] = 0.0

    out_ref[0, 0] += partial

    @pl.when(pl.program_id(0) == num_blocks - 1)
    def _finish():
        out_ref[0, 0] = out_ref[0, 0] * inv_count


def kernel(y_hat, y):
    B, L, K = y_hat.shape
    Bb = 512
    G = B // Bb
    x_t = jnp.transpose(y_hat, (0, 2, 1))
    y3 = y.reshape(G, Bb, 1)
    m = (jnp.arange(Bb * K, dtype=jnp.int32)[None, :] // K
         == jnp.arange(Bb, dtype=jnp.int32)[:, None]).astype(jnp.bfloat16)
    total = pl.pallas_call(
        functools.partial(_loss_body, Bb=Bb, K=K, L=L,
                          inv_count=1.0 / (B * (L - 1)), num_blocks=G),
        grid=(G,),
        in_specs=[
            pl.BlockSpec((Bb, K, L), lambda i: (i, 0, 0)),
            pl.BlockSpec((1, Bb, 1), lambda i: (i, 0, 0)),
            pl.BlockSpec((Bb, Bb * K), lambda i: (0, 0)),
        ],
        out_specs=pl.BlockSpec(memory_space=pltpu.SMEM),
        out_shape=jax.ShapeDtypeStruct((1, 1), jnp.float32),
    )(x_t, y3, m)
    return total[0, 0]


# speedup vs baseline: 1.2535x; 1.1540x over previous
import functools

import jax
import jax.numpy as jnp
from jax.experimental import pallas as pl
from jax.experimental.pallas import tpu as pltpu

_ALPHA = 0.5
_NEG_LAMBDA = 1.0


def _loss_body(x_ref, y_ref, m_ref, out_ref, *, Bb, K, L, CH, inv_count,
               num_blocks):
    x = x_ref[...]
    yb = y_ref[0]

    e16 = jnp.exp(-x).astype(jnp.bfloat16)
    e2 = e16.reshape(Bb * K, L)
    m = m_ref[...]
    s = jnp.concatenate(
        [jax.lax.dot_general(m, e2[c * CH * K:(c + 1) * CH * K],
                             (((1,), (0,)), ((), ())),
                             preferred_element_type=jnp.float32)
         for c in range(Bb // CH)], axis=0)

    col3 = jax.lax.broadcasted_iota(jnp.int32, (Bb, K, L), 2)
    xsel = jnp.where(col3 == yb.reshape(Bb, 1, 1), x, jnp.inf)
    posk = jnp.min(xsel, axis=2)
    pos = jnp.min(posk, axis=1, keepdims=True)

    col = jax.lax.broadcasted_iota(jnp.int32, (Bb, L), 1)
    tmask = col == yb

    t = jnp.maximum(_ALPHA + pos + _NEG_LAMBDA * jnp.log(s), 0.0)
    partial = jnp.sum(jnp.where(tmask, 0.0, t))

    @pl.when(pl.program_id(0) == 0)
    def _init():
        out_ref[0, 0] = 0.0

    out_ref[0, 0] += partial

    @pl.when(pl.program_id(0) == num_blocks - 1)
    def _finish():
        out_ref[0, 0] = out_ref[0, 0] * inv_count


def kernel(y_hat, y):
    B, L, K = y_hat.shape
    Bb = 512
    CH = 128
    G = B // Bb
    x_t = jnp.transpose(y_hat, (0, 2, 1))
    y3 = y.reshape(G, Bb, 1)
    m = (jnp.arange(CH * K, dtype=jnp.int32)[None, :] // K
         == jnp.arange(CH, dtype=jnp.int32)[:, None]).astype(jnp.bfloat16)
    total = pl.pallas_call(
        functools.partial(_loss_body, Bb=Bb, K=K, L=L, CH=CH,
                          inv_count=1.0 / (B * (L - 1)), num_blocks=G),
        grid=(G,),
        in_specs=[
            pl.BlockSpec((Bb, K, L), lambda i: (i, 0, 0)),
            pl.BlockSpec((1, Bb, 1), lambda i: (i, 0, 0)),
            pl.BlockSpec((CH, CH * K), lambda i: (0, 0)),
        ],
        out_specs=pl.BlockSpec(memory_space=pltpu.SMEM),
        out_shape=jax.ShapeDtypeStruct((1, 1), jnp.float32),
    )(x_t, y3, m)
    return total[0, 0]


# speedup vs baseline: 1.2679x; 1.0115x over previous
import functools

import jax
import jax.numpy as jnp
from jax.experimental import pallas as pl
from jax.experimental.pallas import tpu as pltpu

_ALPHA = 0.5
_NEG_LAMBDA = 1.0


def _loss_body(x_ref, y_ref, m_ref, out_ref, *, Bb, K, L, CH, inv_count,
               num_blocks):
    x = x_ref[...]
    yb = y_ref[0]

    e16 = jnp.exp(-x).astype(jnp.bfloat16)
    e2 = e16.reshape(Bb * K, L)
    m = m_ref[...]
    s = jnp.concatenate(
        [jax.lax.dot_general(m, e2[c * CH * K:(c + 1) * CH * K],
                             (((1,), (0,)), ((), ())),
                             preferred_element_type=jnp.float32)
         for c in range(Bb // CH)], axis=0)

    col3 = jax.lax.broadcasted_iota(jnp.int32, (Bb, K, L), 2)
    xsel = jnp.where(col3 == yb.reshape(Bb, 1, 1), x, jnp.inf)
    posk = jnp.min(xsel, axis=2)
    pos = jnp.min(posk, axis=1, keepdims=True)

    col = jax.lax.broadcasted_iota(jnp.int32, (Bb, L), 1)
    tmask = col == yb

    t = jnp.maximum(_ALPHA + pos + _NEG_LAMBDA * jnp.log(s), 0.0)
    partial = jnp.sum(jnp.where(tmask, 0.0, t))

    @pl.when(pl.program_id(0) == 0)
    def _init():
        out_ref[0, 0] = 0.0

    out_ref[0, 0] += partial

    @pl.when(pl.program_id(0) == num_blocks - 1)
    def _finish():
        out_ref[0, 0] = out_ref[0, 0] * inv_count


def kernel(y_hat, y):
    B, L, K = y_hat.shape
    Bb = 512
    CH = 64
    G = B // Bb
    x_t = jnp.transpose(y_hat, (0, 2, 1))
    y3 = y.reshape(G, Bb, 1)
    m = (jnp.arange(CH * K, dtype=jnp.int32)[None, :] // K
         == jnp.arange(CH, dtype=jnp.int32)[:, None]).astype(jnp.bfloat16)
    total = pl.pallas_call(
        functools.partial(_loss_body, Bb=Bb, K=K, L=L, CH=CH,
                          inv_count=1.0 / (B * (L - 1)), num_blocks=G),
        grid=(G,),
        in_specs=[
            pl.BlockSpec((Bb, K, L), lambda i: (i, 0, 0)),
            pl.BlockSpec((1, Bb, 1), lambda i: (i, 0, 0)),
            pl.BlockSpec((CH, CH * K), lambda i: (0, 0)),
        ],
        out_specs=pl.BlockSpec(memory_space=pltpu.SMEM),
        out_shape=jax.ShapeDtypeStruct((1, 1), jnp.float32),
    )(x_t, y3, m)
    return total[0, 0]


# speedup vs baseline: 1.3204x; 1.0414x over previous
import functools

import jax
import jax.numpy as jnp
from jax.experimental import pallas as pl
from jax.experimental.pallas import tpu as pltpu

_ALPHA = 0.5
_NEG_LAMBDA = 1.0


def _loss_body(x_ref, y_ref, m_ref, out_ref, *, Bb, K, L, CH, inv_count,
               num_blocks):
    x = x_ref[...]
    yb = y_ref[0]

    e16 = jnp.exp(-x).astype(jnp.bfloat16)
    e2 = e16.reshape(Bb * K, L)
    m = m_ref[...]
    s = jnp.concatenate(
        [jax.lax.dot_general(m, e2[c * CH * K:(c + 1) * CH * K],
                             (((1,), (0,)), ((), ())),
                             preferred_element_type=jnp.float32)
         for c in range(Bb // CH)], axis=0)

    col3 = jax.lax.broadcasted_iota(jnp.int32, (Bb, K, L), 2)
    xsel = jnp.where(col3 == yb.reshape(Bb, 1, 1), x, jnp.inf)
    posk = jnp.min(xsel, axis=2)
    pos = jnp.min(posk, axis=1, keepdims=True)

    col = jax.lax.broadcasted_iota(jnp.int32, (Bb, L), 1)
    tmask = col == yb

    t = jnp.maximum(_ALPHA + pos + _NEG_LAMBDA * jnp.log(s), 0.0)
    partial = jnp.sum(jnp.where(tmask, 0.0, t))

    @pl.when(pl.program_id(0) == 0)
    def _init():
        out_ref[0, 0] = 0.0

    out_ref[0, 0] += partial

    @pl.when(pl.program_id(0) == num_blocks - 1)
    def _finish():
        out_ref[0, 0] = out_ref[0, 0] * inv_count


def kernel(y_hat, y):
    B, L, K = y_hat.shape
    Bb = 1024
    CH = 64
    G = B // Bb
    x_t = jnp.transpose(y_hat, (0, 2, 1))
    y3 = y.reshape(G, Bb, 1)
    m = (jnp.arange(CH * K, dtype=jnp.int32)[None, :] // K
         == jnp.arange(CH, dtype=jnp.int32)[:, None]).astype(jnp.bfloat16)
    total = pl.pallas_call(
        functools.partial(_loss_body, Bb=Bb, K=K, L=L, CH=CH,
                          inv_count=1.0 / (B * (L - 1)), num_blocks=G),
        grid=(G,),
        in_specs=[
            pl.BlockSpec((Bb, K, L), lambda i: (i, 0, 0)),
            pl.BlockSpec((1, Bb, 1), lambda i: (i, 0, 0)),
            pl.BlockSpec((CH, CH * K), lambda i: (0, 0)),
        ],
        out_specs=pl.BlockSpec(memory_space=pltpu.SMEM),
        out_shape=jax.ShapeDtypeStruct((1, 1), jnp.float32),
    )(x_t, y3, m)
    return total[0, 0]
